# baseline (speedup 1.0000x reference)
import functools

import jax
import jax.numpy as jnp
from jax import lax
from jax.experimental import pallas as pl
from jax.experimental.pallas import tpu as pltpu
from jax.experimental.pallas import tpu_sc as plsc

VOC = 1000000
EMBED = 64
NC = 2
NS = 16
NW = NC * NS
TPW = 6400
NCOL_T = (VOC + 127) // 128
NCOL = 245
DRANGE = NCOL * 128
CAP_UNITS = 96
CAP = CAP_UNITS * 128
SENT_V = 1 << 30
TOTAL = NW * TPW
SENT_P = TOTAL
ROWS_PAD = TOTAL + NW * 128
OUT_ROWS = TOTAL + 128

_TC_ON = pltpu.CompilerParams(use_tc_tiling_on_sc=True,
                              needs_layout_passes=False)


def _iota16():
    return lax.broadcasted_iota(jnp.int32, (16,), 0)


def _scan_bias():
    cnt, _ = plsc.scan_count(jnp.zeros((16,), jnp.int32))
    return jnp.min(cnt)


def _bin_body(xi_hbm, sv_hbm, sp_hbm, slen_hbm, soff_hbm,
              xv, svv, spv, cnt, cur, alen, aoff):
    w = lax.axis_index("s") * NC + lax.axis_index("c")
    pltpu.sync_copy(xi_hbm.at[w], xv)
    z16 = jnp.zeros((16,), jnp.int32)
    cnt[pl.ds(0, 16)] = z16
    cnt[pl.ds(16, 16)] = z16
    bias = _scan_bias()

    def p1(i, _):
        v = xv[pl.ds(i * 16, 16)]
        dd = v // DRANGE
        rank, last = plsc.scan_count(dd)
        base = plsc.load_gather(cnt, [dd])
        plsc.store_scatter(cnt, [dd], base + (rank - bias) + 1, mask=last)
        return ()

    lax.fori_loop(0, TPW // 16, p1, ())

    c16a = cnt[pl.ds(0, 16)]
    csa = plsc.cumsum(c16a)
    offa = csa - c16a
    c16b = cnt[pl.ds(16, 16)]
    csb = plsc.cumsum(c16b)
    offb = csb - c16b + jnp.max(csa)
    cur[pl.ds(0, 16)] = offa
    cur[pl.ds(16, 16)] = offb
    alen[pl.ds(0, 16)] = c16a
    alen[pl.ds(16, 16)] = c16b
    aoff[pl.ds(0, 16)] = offa
    aoff[pl.ds(16, 16)] = offb
    iota = _iota16()

    def p2(i, _):
        v = xv[pl.ds(i * 16, 16)]
        dd = v // DRANGE
        rank, last = plsc.scan_count(dd)
        r0 = rank - bias
        p = (w * TPW + i * 16) + iota
        base = plsc.load_gather(cur, [dd])
        slot = base + r0
        plsc.store_scatter(svv, [slot], v)
        plsc.store_scatter(spv, [slot], p)
        plsc.store_scatter(cur, [dd], base + r0 + 1, mask=last)
        return ()

    lax.fori_loop(0, TPW // 16, p2, ())
    pltpu.sync_copy(svv, sv_hbm.at[w])
    pltpu.sync_copy(spv, sp_hbm.at[w])
    pltpu.sync_copy(alen, slen_hbm.at[w])
    pltpu.sync_copy(aoff, soff_hbm.at[w])


def _totals_and_bases(lenv, bases, pads, d):
    def acc(w, c):
        ta, tb = c
        return (ta + lenv[w, pl.ds(0, 16)], tb + lenv[w, pl.ds(16, 16)])

    z16 = jnp.zeros((16,), jnp.int32)
    tota, totb = lax.fori_loop(0, NW, acc, (z16, z16))
    pa = ((tota + 127) >> 7) << 7
    pb = ((totb + 127) >> 7) << 7
    csa = plsc.cumsum(pa)
    ba = csa - pa
    bb = plsc.cumsum(pb) - pb + jnp.max(csa)
    bases[pl.ds(0, 16)] = ba
    bases[pl.ds(16, 16)] = bb
    pads[pl.ds(0, 16)] = pa
    pads[pl.ds(16, 16)] = pb
    return (bases[pl.ds(d, 16)][0], pads[pl.ds(d, 16)][0],
            tota, totb)


def _stream_body(tokT_hbm, tail_hbm, sv_hbm, sp_hbm, slen_hbm, soff_hbm,
                 posf_hbm, rows_hbm, rp_hbm,
                 lenv, offv, posv, svst, spst, fsv, fsp, fcnt, foff, fcur,
                 bases, pads, colbuf, rstage, pstage, smem, sem_ing, sem_col):
    d = lax.axis_index("s") * NC + lax.axis_index("c")
    lo = d * DRANGE
    hi = lo + DRANGE
    pltpu.sync_copy(slen_hbm, lenv)
    pltpu.sync_copy(soff_hbm, offv)
    pltpu.sync_copy(posf_hbm, posv)
    gbase, _, _, _ = _totals_and_bases(lenv, bases, pads, d)
    bias = _scan_bias()
    iota = _iota16()
    col0 = d * NCOL
    sentp16 = jnp.full((16,), SENT_P, jnp.int32)
    sentv16 = jnp.full((16,), SENT_V, jnp.int32)
    smem[0] = 0
    smem[1] = 0

    def memset_p(i, _):
        pstage[pl.ds(i * 16, 16)] = sentp16
        return ()

    lax.fori_loop(0, 8, memset_p, ())

    def batch(_b, __):
        @pl.when(smem[1] < NW)
        def _():
            def ms(i, _):
                svst[pl.ds(i * 16, 16)] = sentv16
                spst[pl.ds(i * 16, 16)] = sentp16
                return ()

            lax.fori_loop(0, CAP // 16, ms, ())

            ptr = smem[1]
            slot = jnp.int32(0)
            stopped = jnp.int32(0)
            nunits = jnp.int32(0)
            new_ptr = ptr
            for w in range(NW):
                off_w = offv[w, pl.ds(d, 16)][0]
                ln = lenv[w, pl.ds(d, 16)][0]
                u0 = off_w >> 7
                nu = jnp.where(ln > 0, ((off_w + ln + 127) >> 7) - u0, 0)
                sel = (jnp.int32(w) >= ptr) & (stopped == 0)
                fits = (slot + nu) <= CAP_UNITS
                take = sel & fits
                stopped = jnp.where(sel & jnp.logical_not(fits), 1, stopped)
                base_slot = slot

                def ing(u, _, w=w, u0=u0, base_slot=base_slot):
                    src = pl.ds(pl.multiple_of((u0 + u) * 128, 128), 128)
                    dst = pl.ds(pl.multiple_of((base_slot + u) * 128, 128), 128)
                    pltpu.async_copy(sv_hbm.at[w, src], svst.at[dst], sem_ing)
                    pltpu.async_copy(sp_hbm.at[w, src], spst.at[dst], sem_ing)
                    return ()

                @pl.when(take)
                def _(nu=nu, ing=ing):
                    lax.fori_loop(0, nu, ing, ())

                slot = jnp.where(take, slot + nu, slot)
                nunits = jnp.where(take, nunits + nu, nunits)
                new_ptr = jnp.where(take, jnp.int32(w + 1), new_ptr)
            smem[1] = new_ptr

            def drain(_u, __):
                pltpu.make_async_copy(
                    sv_hbm.at[0, pl.ds(0, 128)], svst.at[pl.ds(0, 128)],
                    sem_ing).wait()
                pltpu.make_async_copy(
                    sp_hbm.at[0, pl.ds(0, 128)], spst.at[pl.ds(0, 128)],
                    sem_ing).wait()
                return ()

            lax.fori_loop(0, nunits, drain, ())

            def zf(i, _):
                fcnt[pl.ds(i * 16, 16)] = jnp.zeros((16,), jnp.int32)
                return ()

            lax.fori_loop(0, 16, zf, ())

            def s2a(i, _):
                v = svst[pl.ds(i * 16, 16)]
                valid = (v >= lo) & (v < hi)
                lc = jnp.where(valid, (v - lo) >> 7, NCOL)
                rank, last = plsc.scan_count(lc)
                base = plsc.load_gather(fcnt, [lc])
                plsc.store_scatter(fcnt, [lc], base + (rank - bias) + 1,
                                   mask=last)
                return ()

            lax.fori_loop(0, CAP // 16, s2a, ())

            def pre(i, carry):
                c16 = fcnt[pl.ds(i * 16, 16)]
                cs = plsc.cumsum(c16)
                ex = cs - c16 + carry
                foff[pl.ds(i * 16, 16)] = ex
                fcur[pl.ds(i * 16, 16)] = ex
                return carry + jnp.max(cs)

            lax.fori_loop(0, 16, pre, jnp.int32(0))

            def s2b(i, _):
                v = svst[pl.ds(i * 16, 16)]
                p = spst[pl.ds(i * 16, 16)]
                valid = (v >= lo) & (v < hi)
                lc = jnp.where(valid, (v - lo) >> 7, NCOL)
                pp = jnp.where(valid, p, SENT_P)
                rank, last = plsc.scan_count(lc)
                r0 = rank - bias
                base = plsc.load_gather(fcur, [lc])
                slot2 = base + r0
                plsc.store_scatter(fsv, [slot2], v)
                plsc.store_scatter(fsp, [slot2], pp)
                plsc.store_scatter(fcur, [lc], base + r0 + 1, mask=last)
                return ()

            lax.fori_loop(0, CAP // 16, s2b, ())

            def issue(lc, buf):
                col = col0 + lc

                @pl.when(col < NCOL_T - 1)
                def _():
                    pltpu.async_copy(
                        tokT_hbm.at[:, pl.ds(pl.multiple_of(col * 128, 128),
                                             128)],
                        colbuf.at[buf], sem_col)

                @pl.when(col == NCOL_T - 1)
                def _():
                    pltpu.async_copy(tail_hbm, colbuf.at[buf], sem_col)

            @pl.when(col0 < NCOL_T)
            def _():
                issue(0, 0)

            def s3(lc, _):
                buf = lc & 1
                cv = (col0 + lc) < NCOL_T

                @pl.when(cv)
                def _():
                    pltpu.make_async_copy(
                        tokT_hbm.at[:, pl.ds(0, 128)], colbuf.at[0],
                        sem_col).wait()

                @pl.when((lc + 1 < NCOL) & (col0 + lc + 1 < NCOL_T))
                def _():
                    issue(lc + 1, 1 - buf)

                cvec = foff[pl.ds(lc, 16)]
                c0 = cvec[0]
                c1 = cvec[1]

                def tok(j, _):
                    jj = c0 + j
                    v = fsv[pl.ds(jj, 16)][0]
                    p = fsp[pl.ds(jj, 16)][0]
                    l = v & 127
                    t = lax.rem(p, 200)
                    cnt = smem[0]
                    rrow = cnt & 127
                    lidx = jnp.zeros((16,), jnp.int32) + l
                    for k in range(EMBED // 16):
                        vals = plsc.load_gather(
                            colbuf.at[buf], [iota + 16 * k, lidx])
                        pv = posv[pl.ds(t * EMBED + 16 * k, 16)]
                        rstage[rrow, pl.ds(16 * k, 16)] = vals + pv
                    pbase = (rrow >> 4) << 4
                    lane = rrow - pbase
                    pvec = pstage[pl.ds(pbase, 16)]
                    pstage[pl.ds(pbase, 16)] = jnp.where(iota == lane, p, pvec)
                    ncnt = cnt + 1
                    smem[0] = ncnt

                    @pl.when((ncnt & 127) == 0)
                    def _():
                        br = pl.multiple_of(gbase + ncnt - 128, 128)
                        pltpu.sync_copy(rstage, rows_hbm.at[pl.ds(br, 128)])
                        pltpu.sync_copy(pstage, rp_hbm.at[pl.ds(br, 128)])
                        lax.fori_loop(0, 8, memset_p, ())

                    return ()

                lax.fori_loop(0, c1 - c0, tok, ())
                return ()

            lax.fori_loop(0, NCOL, s3, ())

        return ()

    lax.fori_loop(0, NW, batch, ())

    cnt = smem[0]

    @pl.when((cnt & 127) != 0)
    def _():
        br = pl.multiple_of(gbase + ((cnt >> 7) << 7), 128)
        pltpu.sync_copy(rstage, rows_hbm.at[pl.ds(br, 128)])
        pltpu.sync_copy(pstage, rp_hbm.at[pl.ds(br, 128)])


def _perm_body(rows_hbm, rp_hbm, slen_hbm, out_hbm,
               lenv, bases, pads, wstage, pstage, sem_in, sem_out):
    d = lax.axis_index("s") * NC + lax.axis_index("c")
    pltpu.sync_copy(slen_hbm, lenv)
    gbase, pad_d, _, _ = _totals_and_bases(lenv, bases, pads, d)
    nb = pad_d >> 7

    def issue_in(j, buf):
        base = pl.multiple_of(gbase + j * 128, 128)
        pltpu.async_copy(rows_hbm.at[pl.ds(base, 128)],
                         wstage.at[buf], sem_in)
        pltpu.async_copy(rp_hbm.at[pl.ds(base, 128)],
                         pstage.at[buf], sem_in)

    def wait_in():
        pltpu.make_async_copy(rows_hbm.at[pl.ds(0, 128)], wstage.at[0],
                              sem_in).wait()
        pltpu.make_async_copy(rp_hbm.at[pl.ds(0, 128)], pstage.at[0],
                              sem_in).wait()

    def wait_out(buf):
        pltpu.make_async_copy(wstage.at[buf], out_hbm.at[pstage.at[buf]],
                              sem_out).wait()

    @pl.when(nb > 0)
    def _():
        issue_in(0, 0)

    def blk(j, _):
        buf = j & 1
        wait_in()

        @pl.when(j >= 1)
        def _():
            wait_out(0)

        @pl.when(j + 1 < nb)
        def _():
            issue_in(j + 1, 1 - buf)

        pltpu.async_copy(wstage.at[buf], out_hbm.at[pstage.at[buf]], sem_out)
        return ()

    lax.fori_loop(0, nb, blk, ())

    @pl.when(nb >= 1)
    def _():
        wait_out(0)


def kernel(x, token_emb, pos_emb):
    B, Tcur = x.shape
    xi = x.astype(jnp.int32).reshape(NW, TPW)
    tokT = token_emb.T
    posf = pos_emb[0, :Tcur, :].reshape(-1)
    ntail = VOC - (NCOL_T - 1) * 128
    tailp = jnp.zeros((EMBED, 128), jnp.float32)
    tailp = tailp.at[:, :ntail].set(token_emb[(NCOL_T - 1) * 128:, :].T)

    mesh = plsc.VectorSubcoreMesh(core_axis_name="c", subcore_axis_name="s")
    i32 = jnp.int32
    f32 = jnp.float32

    bin_k = functools.partial(
        pl.kernel,
        out_type=(
            jax.ShapeDtypeStruct((NW, TPW), i32),
            jax.ShapeDtypeStruct((NW, TPW), i32),
            jax.ShapeDtypeStruct((NW, 128), i32),
            jax.ShapeDtypeStruct((NW, 128), i32),
        ),
        mesh=mesh,
        compiler_params=_TC_ON,
        scratch_types=[
            pltpu.VMEM((TPW,), i32), pltpu.VMEM((TPW,), i32),
            pltpu.VMEM((TPW,), i32), pltpu.VMEM((32,), i32),
            pltpu.VMEM((32,), i32), pltpu.VMEM((128,), i32),
            pltpu.VMEM((128,), i32),
        ],
    )(_bin_body)
    sv, sp, slen, soff = bin_k(xi)

    stream_k = functools.partial(
        pl.kernel,
        out_type=(
            jax.ShapeDtypeStruct((ROWS_PAD, 128), f32),
            jax.ShapeDtypeStruct((ROWS_PAD,), i32),
        ),
        mesh=mesh,
        compiler_params=_TC_ON,
        scratch_types=[
            pltpu.VMEM((NW, 128), i32), pltpu.VMEM((NW, 128), i32),
            pltpu.VMEM((Tcur * EMBED,), f32),
            pltpu.VMEM((CAP + 16,), i32), pltpu.VMEM((CAP + 16,), i32),
            pltpu.VMEM((CAP + 16,), i32), pltpu.VMEM((CAP + 16,), i32),
            pltpu.VMEM((272,), i32), pltpu.VMEM((272,), i32),
            pltpu.VMEM((272,), i32),
            pltpu.VMEM((48,), i32), pltpu.VMEM((48,), i32),
            pltpu.VMEM((2, EMBED, 128), f32),
            pltpu.VMEM((128, 128), f32),
            pltpu.VMEM((128,), i32),
            pltpu.SMEM((8,), i32),
            pltpu.SemaphoreType.DMA, pltpu.SemaphoreType.DMA,
        ],
    )(_stream_body)
    rows, rp = stream_k(tokT, tailp, sv, sp, slen, soff, posf)

    perm_k = functools.partial(
        pl.kernel,
        out_type=jax.ShapeDtypeStruct((OUT_ROWS, 128), f32),
        mesh=mesh,
        compiler_params=_TC_ON,
        scratch_types=[
            pltpu.VMEM((NW, 128), i32),
            pltpu.VMEM((48,), i32), pltpu.VMEM((48,), i32),
            pltpu.VMEM((2, 128, 128), f32),
            pltpu.VMEM((2, 128), i32),
            pltpu.SemaphoreType.DMA, pltpu.SemaphoreType.DMA,
        ],
    )(_perm_body)
    out_pad = perm_k(rows, rp, slen)

    return out_pad[:TOTAL, :EMBED].reshape(B, Tcur, EMBED)


# speedup vs baseline: 1.3736x; 1.3736x over previous
import functools

import jax
import jax.numpy as jnp
from jax import lax
from jax.experimental import pallas as pl
from jax.experimental.pallas import tpu as pltpu
from jax.experimental.pallas import tpu_sc as plsc

EMBED = 64
NC = 2
NS = 16
NW = NC * NS
CHUNK = 128
NBUF = 2


def _emb_body(idx_hbm, table_hbm, pos2_hbm, out_hbm,
              idx_v, rows_v, pos_sh, sem_g, sem_s):
    chunks_per_w = idx_hbm.shape[1]
    t = pos2_hbm.shape[0] // 2
    c = lax.axis_index("c")
    s = lax.axis_index("s")
    w = s * NC + c
    pltpu.sync_copy(idx_hbm.at[w], idx_v)

    @pl.when(s == 0)
    def _():
        pltpu.sync_copy(pos2_hbm, pos_sh)
    plsc.subcore_barrier()

    base = w * (chunks_per_w * CHUNK)

    def chunk_body(h, _):
        buf = lax.rem(h, NBUF)

        @pl.when(h >= NBUF)
        def _():
            pltpu.make_async_copy(
                rows_v.at[buf], out_hbm.at[pl.ds(base, CHUNK)], sem_s
            ).wait()

        poff = pl.multiple_of(lax.rem(h * CHUNK, t), 8)
        pltpu.sync_copy(pos_sh.at[pl.ds(poff, CHUNK)], rows_v.at[buf])
        pltpu.async_copy(
            table_hbm.at[idx_v.at[h]], rows_v.at[buf], sem_g, add=True
        ).wait()
        pltpu.async_copy(
            rows_v.at[buf],
            out_hbm.at[pl.ds(pl.multiple_of(base + h * CHUNK, CHUNK), CHUNK)],
            sem_s,
        )
        return ()

    lax.fori_loop(0, chunks_per_w, chunk_body, ())

    for _ in range(NBUF):
        pltpu.make_async_copy(
            rows_v.at[0], out_hbm.at[pl.ds(base, CHUNK)], sem_s
        ).wait()


def kernel(x, token_emb, pos_emb):
    B, Tcur = x.shape
    total = B * Tcur
    chunks_per_w = total // (NW * CHUNK)
    xi = x.astype(jnp.int32).reshape(NW, chunks_per_w, CHUNK)
    tpad = jnp.pad(token_emb, ((0, 0), (0, 128 - EMBED)))
    pos = pos_emb[0, :Tcur, :]
    pos2 = jnp.pad(jnp.concatenate([pos, pos], axis=0),
                   ((0, 0), (0, 128 - EMBED)))

    emb = functools.partial(
        pl.kernel,
        out_type=jax.ShapeDtypeStruct((total, 128), jnp.float32),
        mesh=plsc.VectorSubcoreMesh(core_axis_name="c", subcore_axis_name="s"),
        compiler_params=pltpu.CompilerParams(use_tc_tiling_on_sc=True,
                                             needs_layout_passes=False),
        scratch_types=[
            pltpu.VMEM((chunks_per_w, CHUNK), jnp.int32),
            pltpu.VMEM((NBUF, CHUNK, 128), jnp.float32),
            pltpu.VMEM_SHARED((2 * Tcur, 128), jnp.float32),
            pltpu.SemaphoreType.DMA,
            pltpu.SemaphoreType.DMA,
        ],
    )(_emb_body)
    out = emb(xi, tpad, pos2)
    return out[:, :EMBED].reshape(B, Tcur, EMBED)


# speedup vs baseline: 1.4092x; 1.0259x over previous
import functools

import jax
import jax.numpy as jnp
from jax import lax
from jax.experimental import pallas as pl
from jax.experimental.pallas import tpu as pltpu
from jax.experimental.pallas import tpu_sc as plsc

EMBED = 64
NC = 2
NS = 16
NW = NC * NS
CHUNK = 128
NBUF = 3


def _emb_body(idx_hbm, table_hbm, pos2_hbm, out_hbm,
              idx_v, rows_v, pos_sh, sem_g, sem_s, sem_p):
    chunks_per_w = idx_hbm.shape[1]
    t = pos2_hbm.shape[0] // 2
    c = lax.axis_index("c")
    s = lax.axis_index("s")
    w = s * NC + c
    pltpu.sync_copy(idx_hbm.at[w], idx_v)

    @pl.when(s == 0)
    def _():
        pltpu.sync_copy(pos2_hbm, pos_sh)
    plsc.subcore_barrier()

    base = w * (chunks_per_w * CHUNK)

    def prefill(h):
        poff = pl.multiple_of(lax.rem(h * CHUNK, t), 8)
        pltpu.async_copy(pos_sh.at[pl.ds(poff, CHUNK)],
                         rows_v.at[lax.rem(h, NBUF)], sem_p)

    def wait_prefill():
        pltpu.make_async_copy(pos_sh.at[pl.ds(0, CHUNK)], rows_v.at[0],
                              sem_p).wait()

    def wait_store():
        pltpu.make_async_copy(rows_v.at[0], out_hbm.at[pl.ds(base, CHUNK)],
                              sem_s).wait()

    prefill(0)
    prefill(1)

    def chunk_body(h, _):
        buf = lax.rem(h, NBUF)
        wait_prefill()
        pltpu.async_copy(
            table_hbm.at[idx_v.at[h]], rows_v.at[buf], sem_g, add=True
        ).wait()
        pltpu.async_copy(
            rows_v.at[buf],
            out_hbm.at[pl.ds(pl.multiple_of(base + h * CHUNK, CHUNK), CHUNK)],
            sem_s,
        )

        @pl.when(h + 2 < chunks_per_w)
        def _():
            @pl.when(h >= 1)
            def _():
                wait_store()
            prefill(h + 2)

        return ()

    lax.fori_loop(0, chunks_per_w, chunk_body, ())

    for _ in range(NBUF):
        wait_store()


def kernel(x, token_emb, pos_emb):
    B, Tcur = x.shape
    total = B * Tcur
    chunks_per_w = total // (NW * CHUNK)
    xi = x.astype(jnp.int32).reshape(NW, chunks_per_w, CHUNK)
    tpad = jnp.pad(token_emb, ((0, 0), (0, 128 - EMBED)))
    pos = pos_emb[0, :Tcur, :]
    pos2 = jnp.pad(jnp.concatenate([pos, pos], axis=0),
                   ((0, 0), (0, 128 - EMBED)))

    emb = functools.partial(
        pl.kernel,
        out_type=jax.ShapeDtypeStruct((total, 128), jnp.float32),
        mesh=plsc.VectorSubcoreMesh(core_axis_name="c", subcore_axis_name="s"),
        compiler_params=pltpu.CompilerParams(use_tc_tiling_on_sc=True,
                                             needs_layout_passes=False),
        scratch_types=[
            pltpu.VMEM((chunks_per_w, CHUNK), jnp.int32),
            pltpu.VMEM((NBUF, CHUNK, 128), jnp.float32),
            pltpu.VMEM_SHARED((2 * Tcur, 128), jnp.float32),
            pltpu.SemaphoreType.DMA,
            pltpu.SemaphoreType.DMA,
            pltpu.SemaphoreType.DMA,
        ],
    )(_emb_body)
    out = emb(xi, tpad, pos2)
    return out[:, :EMBED].reshape(B, Tcur, EMBED)
